# baseline (speedup 1.0000x reference)
import functools

import jax
import jax.numpy as jnp
from jax import lax
from jax.experimental import pallas as pl
from jax.experimental.pallas import tpu as pltpu
from jax.experimental.pallas import tpu_sc as plsc

_NC = 2
_NS = 16
_NW = _NC * _NS
_L = 16


def _iota16():
    return lax.iota(jnp.int32, 16)


@functools.partial(jax.jit, static_argnums=())
def _pack(tabT, tail2):
    d, v = tabT.shape
    n_full = v // 128
    n_tail = v - n_full * 128
    per_w = n_full // _NW
    n_rem = n_full - per_w * _NW
    n2 = per_w // 2

    mesh = plsc.VectorSubcoreMesh(core_axis_name="c", subcore_axis_name="s")

    @functools.partial(
        pl.kernel,
        out_type=jax.ShapeDtypeStruct((v // 2, 2 * d), jnp.float32),
        mesh=mesh,
        compiler_params=pltpu.CompilerParams(use_tc_tiling_on_sc=True, needs_layout_passes=False),
        scratch_types=[
            pltpu.VMEM((d, 128), jnp.float32),
            pltpu.VMEM((d, 128), jnp.float32),
            pltpu.VMEM((d, 128), jnp.float32),
            pltpu.VMEM((d, 128), jnp.float32),
            pltpu.SemaphoreType.DMA,
            pltpu.SemaphoreType.DMA,
            pltpu.SemaphoreType.DMA,
            pltpu.SemaphoreType.DMA,
        ],
    )
    def body(tab_hbm, tail_hbm, out_hbm, in0, in1, ot0, ot1, g0, g1, s0, s1):
        ins = (in0, in1)
        ots = (ot0, ot1)
        gsems = (g0, g1)
        ssems = (s0, s1)
        wid = lax.axis_index("s") * _NC + lax.axis_index("c")
        iot = _iota16()

        def fire_in(blk, b):
            pltpu.async_copy(
                tab_hbm.at[:, pl.ds(blk * 128, 128)], ins[b], gsems[b]
            )

        def drain_in(b):
            pltpu.make_async_copy(
                tab_hbm.at[:, pl.ds(0, 128)], ins[b], gsems[b]
            ).wait()

        def fire_store(blk, b, nrow):
            pltpu.async_copy(
                ots[b].at[pl.ds(0, nrow), :] if nrow < d else ots[b],
                out_hbm.at[pl.ds(blk * 64, nrow), :],
                ssems[b],
            )

        def wait_store(b, nrow):
            pltpu.make_async_copy(
                ots[b].at[pl.ds(0, nrow), :] if nrow < d else ots[b],
                out_hbm.at[pl.ds(0, nrow), :],
                ssems[b],
            ).wait()

        def transpose(b, nq):
            @plsc.parallel_loop(0, nq, unroll=2)
            def _(q):
                for m in range(8):
                    col = 2 * q + (m // 4)
                    src = plsc.load_gather(
                        ins[b],
                        [iot + 16 * (m % 4), jnp.zeros((16,), jnp.int32) + col],
                    )
                    ots[b][q, pl.ds(16 * m, 16)] = src

        def blk_of(i):
            return wid + _NW * i

        fire_in(blk_of(0), 0)

        def jbody(j, carry):
            fire_in(blk_of(2 * j + 1), 1)
            drain_in(0)

            @pl.when(j > 0)
            def _():
                wait_store(0, d)

            transpose(0, d)
            fire_store(blk_of(2 * j), 0, d)

            @pl.when(j < n2 - 1)
            def _():
                fire_in(blk_of(2 * j + 2), 0)

            drain_in(1)

            @pl.when(j > 0)
            def _():
                wait_store(1, d)

            transpose(1, d)
            fire_store(blk_of(2 * j + 1), 1, d)
            return carry

        lax.fori_loop(0, n2, jbody, 0)
        wait_store(0, d)
        wait_store(1, d)

        @pl.when(wid < n_rem)
        def _():
            blk = n_full - n_rem + wid
            pltpu.async_copy(tab_hbm.at[:, pl.ds(blk * 128, 128)], in0, g0)
            drain_in(0)
            transpose(0, d)
            fire_store(blk, 0, d)
            wait_store(0, d)

        if n_tail:
            @pl.when(wid == n_rem)
            def _():
                nr = n_tail // 2
                pltpu.sync_copy(tail_hbm, in1.at[pl.ds(0, nr), :])
                pltpu.sync_copy(
                    in1.at[pl.ds(0, nr), :],
                    out_hbm.at[pl.ds(n_full * 64, nr), :],
                )

    return body(tabT, tail2)


@functools.partial(jax.jit, static_argnums=())
def _gather(xT, packed, pos_flat):
    s_len, b_len = xT.shape
    d = 64
    n_sb = s_len // 8

    mesh = plsc.VectorSubcoreMesh(core_axis_name="c", subcore_axis_name="s")

    @functools.partial(
        pl.kernel,
        out_type=jax.ShapeDtypeStruct((s_len, d, b_len), jnp.float32),
        mesh=mesh,
        compiler_params=pltpu.CompilerParams(use_tc_tiling_on_sc=True, needs_layout_passes=False),
        scratch_types=[
            pltpu.VMEM((8, 128), jnp.int32),
            pltpu.VMEM((8, 128), jnp.int32),
            pltpu.VMEM((8, 128), jnp.int32),
            pltpu.VMEM((128, 2 * d), jnp.float32),
            pltpu.VMEM((128, 2 * d), jnp.float32),
            pltpu.VMEM((d, 128), jnp.float32),
            pltpu.VMEM((d, 128), jnp.float32),
            pltpu.VMEM((s_len * d,), jnp.float32),
            pltpu.SemaphoreType.DMA,
            pltpu.SemaphoreType.DMA,
            pltpu.SemaphoreType.DMA,
            pltpu.SemaphoreType.DMA,
        ],
    )
    def body(x_hbm, tab_hbm, pos_hbm, out_hbm, xblk, idx2, off2,
             r0, r1, sl0, sl1, pos_v, g0, g1, s0, s1):
        rows = (r0, r1)
        slabs = (sl0, sl1)
        gsems = (g0, g1)
        ssems = (s0, s1)
        wid = lax.axis_index("s") * _NC + lax.axis_index("c")
        b0 = wid * 128
        iot = _iota16()
        pltpu.sync_copy(pos_hbm, pos_v)

        def fire_gather(sr, b):
            pltpu.async_copy(tab_hbm.at[idx2.at[sr]], rows[b], gsems[b])

        def drain_gather(b):
            pltpu.make_async_copy(
                tab_hbm.at[pl.ds(0, 128), :], rows[b], gsems[b]
            ).wait()

        def fire_slab(s, b):
            pltpu.async_copy(
                slabs[b], out_hbm.at[s, :, pl.ds(b0, 128)], ssems[b]
            )

        def wait_slab(b):
            pltpu.make_async_copy(
                slabs[b], out_hbm.at[0, :, pl.ds(0, 128)], ssems[b]
            ).wait()

        def sb_body(sb, carry):
            pltpu.sync_copy(
                x_hbm.at[pl.ds(sb * 8, 8), pl.ds(b0, 128)], xblk
            )

            @plsc.parallel_loop(0, 8)
            def _(sr):
                for g in range(8):
                    xv = xblk[sr, pl.ds(16 * g, 16)]
                    idx2[sr, pl.ds(16 * g, 16)] = lax.shift_right_logical(
                        xv, 1
                    )
                    off2[sr, pl.ds(16 * g, 16)] = lax.shift_left(xv & 1, 6)

            fire_gather(0, 0)
            for sr in range(8):
                b = sr % 2
                if sr < 7:
                    fire_gather(sr + 1, 1 - b)
                drain_gather(b)

                @pl.when((sb > 0) | (sr >= 2))
                def _():
                    wait_slab(b)

                s = sb * 8 + sr
                offv = tuple(off2[sr, pl.ds(16 * g, 16)] for g in range(8))

                @plsc.parallel_loop(0, d, unroll=2, carry=offv)
                def _(e, ofs):
                    pv = plsc.load_gather(
                        pos_v, [jnp.zeros((16,), jnp.int32) + (s * d + e)]
                    )
                    for g in range(8):
                        tv = plsc.load_gather(
                            rows[b], [iot + 16 * g, ofs[g] + e]
                        )
                        slabs[b][e, pl.ds(16 * g, 16)] = tv + pv
                    return ofs

                fire_slab(s, b)
            return carry

        lax.fori_loop(0, n_sb, sb_body, 0)
        wait_slab(0)
        wait_slab(1)

    return body(xT, packed, pos_flat)


def kernel(x, token_table, pos_table):
    b, s = x.shape
    v, d = token_table.shape
    xT = x.T.astype(jnp.int32)
    tabT = token_table.T
    pos_flat = pos_table[:s].reshape(s * d)
    n_tail = v % 128
    tail2 = token_table[v - n_tail:].reshape(n_tail // 2, 2 * d)
    packed = _pack(tabT, tail2)
    outP = _gather(xT, packed, pos_flat)
    return outP.transpose(2, 0, 1)


# speedup vs baseline: 1.2299x; 1.2299x over previous
import functools

import jax
import jax.numpy as jnp
from jax import lax
from jax.experimental import pallas as pl
from jax.experimental.pallas import tpu as pltpu
from jax.experimental.pallas import tpu_sc as plsc

_NC = 2
_NS = 16
_NW = _NC * _NS
_L = 16

_SUB = 80
_NSUB = 10
_CHUNK = _SUB * _NSUB


@functools.partial(jax.jit, static_argnums=(3, 4))
def _emb(x_flat, token_table, pos_flat, seq, d):
    n_rows = x_flat.shape[0]
    n_w = n_rows // _NW
    n_chunks = n_w // _CHUNK
    n2 = n_chunks // 2
    reps = _CHUNK // seq
    dvr = d // _L

    mesh = plsc.VectorSubcoreMesh(core_axis_name="c", subcore_axis_name="s")

    @functools.partial(
        pl.kernel,
        out_type=jax.ShapeDtypeStruct((n_rows, d), jnp.float32),
        mesh=mesh,
        compiler_params=pltpu.CompilerParams(use_tc_tiling_on_sc=False),
        scratch_types=[
            pltpu.VMEM((_CHUNK,), jnp.int32),
            pltpu.VMEM((_CHUNK,), jnp.int32),
            pltpu.VMEM((_CHUNK, d), jnp.float32),
            pltpu.VMEM((_CHUNK, d), jnp.float32),
            pltpu.VMEM((seq * d,), jnp.float32),
            pltpu.SemaphoreType.DMA,
            pltpu.SemaphoreType.DMA,
            pltpu.SemaphoreType.DMA,
            pltpu.SemaphoreType.DMA,
        ],
    )
    def body(x_hbm, tab_hbm, pos_hbm, out_hbm, idx0, idx1, rows0, rows1,
             pos_v, gsem0, gsem1, ssem0, ssem1):
        idxs = (idx0, idx1)
        rows = (rows0, rows1)
        gsems = (gsem0, gsem1)
        ssems = (ssem0, ssem1)

        wid = lax.axis_index("s") * _NC + lax.axis_index("c")
        base = wid * n_w
        pltpu.sync_copy(pos_hbm, pos_v)

        def fire_gather(chunk_i, b):
            off = base + chunk_i * _CHUNK
            pltpu.sync_copy(x_hbm.at[pl.ds(off, _CHUNK)], idxs[b])
            for j in range(_NSUB):
                pltpu.async_copy(
                    tab_hbm.at[idxs[b].at[pl.ds(j * _SUB, _SUB)]],
                    rows[b].at[pl.ds(j * _SUB, _SUB)],
                    gsems[b],
                )

        def drain_gather(b):
            pltpu.make_async_copy(
                out_hbm.at[pl.ds(0, _CHUNK)], rows[b], gsems[b]
            ).wait()

        def fire_store(chunk_i, b):
            off = base + chunk_i * _CHUNK
            pltpu.async_copy(rows[b], out_hbm.at[pl.ds(off, _CHUNK)], ssems[b])

        def wait_store(b):
            pltpu.make_async_copy(
                rows[b], out_hbm.at[pl.ds(0, _CHUNK)], ssems[b]
            ).wait()

        def add_pos(b):
            @plsc.parallel_loop(0, seq, unroll=4)
            def _(s):
                for dd in range(dvr):
                    pv = pos_v[pl.ds(s * d + dd * _L, _L)]
                    for q in range(reps):
                        plsc.addupdate(
                            rows[b].at[q * seq + s, pl.ds(dd * _L, _L)], pv
                        )

        fire_gather(0, 0)

        def jbody(j, carry):
            @pl.when(j > 0)
            def _():
                wait_store(1)

            fire_gather(2 * j + 1, 1)
            drain_gather(0)
            add_pos(0)
            fire_store(2 * j, 0)

            @pl.when(j < n2 - 1)
            def _():
                wait_store(0)
                fire_gather(2 * j + 2, 0)

            drain_gather(1)
            add_pos(1)
            fire_store(2 * j + 1, 1)
            return carry

        lax.fori_loop(0, n2, jbody, 0)
        wait_store(0)
        wait_store(1)

    return body(x_flat, token_table, pos_flat)


def kernel(x, token_table, pos_table):
    b, s = x.shape
    v, d = token_table.shape
    n = b * s
    x_flat = x.reshape(n).astype(jnp.int32)
    pos_flat = pos_table[:s].reshape(s * d)
    out = _emb(x_flat, token_table, pos_flat, s, d)
    return out.reshape(b, s, d)
